# baseline (speedup 1.0000x reference)
import functools

import jax
import jax.numpy as jnp
from jax import lax
from jax.experimental import pallas as pl
from jax.experimental.pallas import tpu as pltpu
from jax.experimental.pallas import tpu_sc as plsc

NC = 2
NS = 16
NW = NC * NS
LANES = 16

NB_ROWS = 1000000
BATCH = 16384
DIM = 64
CHUNKS = DIM // LANES
ROWS_PER_W = BATCH // NW
NBUF = 8
BLK = 128
STG = 128
NCHUNK = ROWS_PER_W // STG

_GATHER_DNUMS = lax.GatherDimensionNumbers(
    offset_dims=(), collapsed_slice_dims=(0,), start_index_map=(0,))


def _xlane(v, idx):
    return lax.gather(v, idx[:, None], _GATHER_DNUMS, slice_sizes=(1,),
                      mode=lax.GatherScatterMode.PROMISE_IN_BOUNDS)


def _rsqrt(ss):
    i = lax.bitcast_convert_type(ss, jnp.int32)
    i = jnp.int32(0x5F3759DF) - (i >> 1)
    y = lax.bitcast_convert_type(i, jnp.float32)
    ssh = 0.5 * ss
    for _ in range(3):
        y = y * (1.5 - ssh * y * y)
    return y


def _mesh():
    return plsc.VectorSubcoreMesh(core_axis_name="c", subcore_axis_name="s")


@functools.partial(
    pl.kernel,
    mesh=_mesh(),
    out_type=[
        jax.ShapeDtypeStruct((DIM, BATCH), jnp.float32),
        jax.ShapeDtypeStruct((DIM, BATCH), jnp.float32),
    ],
    compiler_params=pltpu.CompilerParams(use_tc_tiling_on_sc=True,
                                         needs_layout_passes=False),
    scratch_types=[
        pltpu.VMEM((ROWS_PER_W + LANES,), jnp.int32),
        pltpu.VMEM((NBUF, DIM, BLK), jnp.float32),
        pltpu.VMEM((2, DIM, STG), jnp.float32),
    ] + [pltpu.SemaphoreType.DMA] * (NBUF + 2),
)
def _embed_norm(user_ids, item_ids, user_table_t, item_table_t,
                out_u, out_i, idx_v, ring, staging, *sems):
    ring_sems = sems[:NBUF]
    out_sems = sems[NBUF:]
    wid = lax.axis_index("s") * NC + lax.axis_index("c")
    base = wid * ROWS_PER_W
    lanes = lax.iota(jnp.int32, LANES)
    fidx = [lanes + c * LANES for c in range(CHUNKS)]

    def run_table(tab_t, ids, out):
        pltpu.sync_copy(ids.at[pl.ds(base, ROWS_PER_W)],
                        idx_v.at[pl.ds(0, ROWS_PER_W)])

        def read_idx(j):
            return idx_v[pl.ds(j, LANES)][0]

        def blk_start(iv):
            return pl.multiple_of(iv & jnp.int32(~(BLK - 1)), BLK)

        def fire(j, slot):
            start = blk_start(read_idx(j))
            return pltpu.async_copy(
                tab_t.at[:, pl.ds(start, BLK)], ring.at[slot],
                ring_sems[slot])

        def process(j, slot, bank):
            iv = read_idx(j)
            colv = jnp.full((LANES,), iv & jnp.int32(BLK - 1), jnp.int32)
            x = [plsc.load_gather(ring.at[slot], [fidx[c], colv])
                 for c in range(CHUNKS)]
            p = x[0] * x[0]
            for c in range(1, CHUNKS):
                p = p + x[c] * x[c]
            for sh in (8, 4, 2, 1):
                p = p + _xlane(p, lanes ^ sh)
            y = _rsqrt(jnp.maximum(p, 1e-30))
            jl = j % STG if isinstance(j, int) else j & jnp.int32(STG - 1)
            jlv = jnp.full((LANES,), jl, jnp.int32)
            for c in range(CHUNKS):
                plsc.store_scatter(staging.at[bank], [fidx[c], jlv],
                                   x[c] * y)

        waits = [fire(jnp.int32(s), s) for s in range(NBUF)]
        out_waits = [None, None]
        for k in range(NCHUNK):
            bank = k % 2
            if out_waits[bank] is not None:
                out_waits[bank].wait()

            def group(g, carry, k=k, bank=bank):
                for s in range(NBUF):
                    j = jnp.int32(k * STG) + g * NBUF + s
                    waits[s].wait()
                    process(j, s, bank)
                    fire(j + NBUF, s)
                return carry
            last = (k == NCHUNK - 1)
            ngrp = STG // NBUF - (1 if last else 0)
            lax.fori_loop(0, ngrp, group, 0)
            if last:
                for s in range(NBUF):
                    j = k * STG + (STG // NBUF - 1) * NBUF + s
                    waits[s].wait()
                    process(j, s, bank)
            out_waits[bank] = pltpu.async_copy(
                staging.at[bank], out.at[:, pl.ds(base + k * STG, STG)],
                out_sems[bank])
        for w in out_waits:
            w.wait()

    run_table(user_table_t, user_ids, out_u)
    run_table(item_table_t, item_ids, out_i)


def kernel(user_ids, item_ids, user_table, item_table):
    out_u_t, out_i_t = _embed_norm(user_ids.astype(jnp.int32),
                                   item_ids.astype(jnp.int32),
                                   user_table.T, item_table.T)
    return out_u_t.T, out_i_t.T


# speedup vs baseline: 1.0016x; 1.0016x over previous
import functools

import jax
import jax.numpy as jnp
from jax import lax
from jax.experimental import pallas as pl
from jax.experimental.pallas import tpu as pltpu
from jax.experimental.pallas import tpu_sc as plsc

NC = 2
NS = 16
NW = NC * NS
LANES = 16

NB_ROWS = 1000000
BATCH = 16384
DIM = 64
CHUNKS = DIM // LANES
ROWS_PER_W = BATCH // NW
NBUF = 8
BLK = 128
STG = 128
NCHUNK = ROWS_PER_W // STG

_GATHER_DNUMS = lax.GatherDimensionNumbers(
    offset_dims=(), collapsed_slice_dims=(0,), start_index_map=(0,))


def _xlane(v, idx):
    return lax.gather(v, idx[:, None], _GATHER_DNUMS, slice_sizes=(1,),
                      mode=lax.GatherScatterMode.PROMISE_IN_BOUNDS)


def _rsqrt(ss):
    i = lax.bitcast_convert_type(ss, jnp.int32)
    i = jnp.int32(0x5F3759DF) - (i >> 1)
    y = lax.bitcast_convert_type(i, jnp.float32)
    ssh = 0.5 * ss
    for _ in range(3):
        y = y * (1.5 - ssh * y * y)
    return y


def _mesh():
    return plsc.VectorSubcoreMesh(core_axis_name="c", subcore_axis_name="s")


@functools.partial(
    pl.kernel,
    mesh=_mesh(),
    out_type=[
        jax.ShapeDtypeStruct((DIM, BATCH), jnp.float32),
        jax.ShapeDtypeStruct((DIM, BATCH), jnp.float32),
    ],
    compiler_params=pltpu.CompilerParams(use_tc_tiling_on_sc=True,
                                         needs_layout_passes=False),
    scratch_types=[
        pltpu.VMEM((ROWS_PER_W + LANES,), jnp.int32),
        pltpu.VMEM((NBUF, DIM, BLK), jnp.float32),
        pltpu.VMEM((2, DIM, STG), jnp.float32),
    ] + [pltpu.SemaphoreType.DMA] * (NBUF + 2),
)
def _embed_norm(user_ids, item_ids, user_table_t, item_table_t,
                out_u, out_i, idx_v, ring, staging, *sems):
    ring_sems = sems[:NBUF]
    out_sems = sems[NBUF:]
    wid = lax.axis_index("s") * NC + lax.axis_index("c")
    base = wid * ROWS_PER_W
    lanes = lax.iota(jnp.int32, LANES)
    fidx = [lanes + c * LANES for c in range(CHUNKS)]

    def run_table(tab_t, ids, out):
        pltpu.sync_copy(ids.at[pl.ds(base, ROWS_PER_W)],
                        idx_v.at[pl.ds(0, ROWS_PER_W)])

        def read_idx(j):
            return idx_v[pl.ds(j, LANES)][0]

        def blk_start(iv):
            return pl.multiple_of(iv & jnp.int32(~(BLK - 1)), BLK)

        def fire(j, slot):
            start = blk_start(read_idx(j))
            w1 = pltpu.async_copy(
                tab_t.at[pl.ds(0, DIM // 2), pl.ds(start, BLK)],
                ring.at[slot, pl.ds(0, DIM // 2)], ring_sems[slot])
            w2 = pltpu.async_copy(
                tab_t.at[pl.ds(DIM // 2, DIM // 2), pl.ds(start, BLK)],
                ring.at[slot, pl.ds(DIM // 2, DIM // 2)], ring_sems[slot])
            return (w1, w2)

        def process(j, slot, bank):
            iv = read_idx(j)
            colv = jnp.full((LANES,), iv & jnp.int32(BLK - 1), jnp.int32)
            x = [plsc.load_gather(ring.at[slot], [fidx[c], colv])
                 for c in range(CHUNKS)]
            p = x[0] * x[0]
            for c in range(1, CHUNKS):
                p = p + x[c] * x[c]
            for sh in (8, 4, 2, 1):
                p = p + _xlane(p, lanes ^ sh)
            y = _rsqrt(jnp.maximum(p, 1e-30))
            jl = j % STG if isinstance(j, int) else j & jnp.int32(STG - 1)
            jlv = jnp.full((LANES,), jl, jnp.int32)
            for c in range(CHUNKS):
                plsc.store_scatter(staging.at[bank], [fidx[c], jlv],
                                   x[c] * y)

        waits = [fire(jnp.int32(s), s) for s in range(NBUF)]
        out_waits = [None, None]
        for k in range(NCHUNK):
            bank = k % 2
            if out_waits[bank] is not None:
                out_waits[bank].wait()

            def group(g, carry, k=k, bank=bank):
                for s in range(NBUF):
                    j = jnp.int32(k * STG) + g * NBUF + s
                    for w in waits[s]:
                        w.wait()
                    process(j, s, bank)
                    fire(j + NBUF, s)
                return carry
            last = (k == NCHUNK - 1)
            ngrp = STG // NBUF - (1 if last else 0)
            lax.fori_loop(0, ngrp, group, 0)
            if last:
                for s in range(NBUF):
                    j = k * STG + (STG // NBUF - 1) * NBUF + s
                    for w in waits[s]:
                        w.wait()
                    process(j, s, bank)
            out_waits[bank] = pltpu.async_copy(
                staging.at[bank], out.at[:, pl.ds(base + k * STG, STG)],
                out_sems[bank])
        for w in out_waits:
            w.wait()

    run_table(user_table_t, user_ids, out_u)
    run_table(item_table_t, item_ids, out_i)


def kernel(user_ids, item_ids, user_table, item_table):
    out_u_t, out_i_t = _embed_norm(user_ids.astype(jnp.int32),
                                   item_ids.astype(jnp.int32),
                                   user_table.T, item_table.T)
    return out_u_t.T, out_i_t.T


# speedup vs baseline: 1.0031x; 1.0016x over previous
import functools

import jax
import jax.numpy as jnp
from jax import lax
from jax.experimental import pallas as pl
from jax.experimental.pallas import tpu as pltpu
from jax.experimental.pallas import tpu_sc as plsc

NC = 2
NS = 16
NW = NC * NS
LANES = 16

NB_ROWS = 1000000
BATCH = 16384
DIM = 64
CHUNKS = DIM // LANES
ROWS_PER_W = BATCH // NW
NBUF = 8
BLK = 128
STG = 128
NCHUNK = ROWS_PER_W // STG

_GATHER_DNUMS = lax.GatherDimensionNumbers(
    offset_dims=(), collapsed_slice_dims=(0,), start_index_map=(0,))


def _xlane(v, idx):
    return lax.gather(v, idx[:, None], _GATHER_DNUMS, slice_sizes=(1,),
                      mode=lax.GatherScatterMode.PROMISE_IN_BOUNDS)


def _rsqrt(ss):
    i = lax.bitcast_convert_type(ss, jnp.int32)
    i = jnp.int32(0x5F3759DF) - (i >> 1)
    y = lax.bitcast_convert_type(i, jnp.float32)
    ssh = 0.5 * ss
    for _ in range(3):
        y = y * (1.5 - ssh * y * y)
    return y


def _mesh():
    return plsc.VectorSubcoreMesh(core_axis_name="c", subcore_axis_name="s")


@functools.partial(
    pl.kernel,
    mesh=_mesh(),
    out_type=[
        jax.ShapeDtypeStruct((DIM, BATCH), jnp.float32),
        jax.ShapeDtypeStruct((DIM, BATCH), jnp.float32),
    ],
    compiler_params=pltpu.CompilerParams(use_tc_tiling_on_sc=True,
                                         needs_layout_passes=False),
    scratch_types=[
        pltpu.VMEM((ROWS_PER_W + LANES,), jnp.int32),
        pltpu.VMEM((NBUF, DIM, BLK), jnp.float32),
        pltpu.VMEM((2, DIM, STG), jnp.float32),
    ] + [pltpu.SemaphoreType.DMA] * (NBUF + 2),
)
def _embed_norm(user_ids, item_ids, user_table_t, item_table_t,
                out_u, out_i, idx_v, ring, staging, *sems):
    ring_sems = sems[:NBUF]
    out_sems = sems[NBUF:]
    wid = lax.axis_index("s") * NC + lax.axis_index("c")
    base = wid * ROWS_PER_W
    lanes = lax.iota(jnp.int32, LANES)
    fidx = [lanes + c * LANES for c in range(CHUNKS)]

    def run_table(tab_t, ids, out):
        pltpu.sync_copy(ids.at[pl.ds(base, ROWS_PER_W)],
                        idx_v.at[pl.ds(0, ROWS_PER_W)])

        def read_idx(j):
            return idx_v[pl.ds(j, LANES)][0]

        def blk_start(iv):
            return pl.multiple_of(iv & jnp.int32(~(BLK - 1)), BLK)

        def fire(j, slot):
            start = blk_start(read_idx(j))
            return pltpu.async_copy(
                tab_t.at[:, pl.ds(start, BLK)], ring.at[slot],
                ring_sems[slot])

        def process(j, slot, bank):
            iv = read_idx(j)
            colv = jnp.full((LANES,), iv & jnp.int32(BLK - 1), jnp.int32)
            x = [plsc.load_gather(ring.at[slot], [fidx[c], colv])
                 for c in range(CHUNKS)]
            p = x[0] * x[0]
            for c in range(1, CHUNKS):
                p = p + x[c] * x[c]
            for sh in (8, 4, 2, 1):
                p = p + _xlane(p, lanes ^ sh)
            y = _rsqrt(jnp.maximum(p, 1e-24))
            jl = j % STG if isinstance(j, int) else j & jnp.int32(STG - 1)
            jlv = jnp.full((LANES,), jl, jnp.int32)
            for c in range(CHUNKS):
                plsc.store_scatter(staging.at[bank], [fidx[c], jlv],
                                   x[c] * y)

        waits = [fire(jnp.int32(s), s) for s in range(NBUF)]
        out_waits = [None, None]
        for k in range(NCHUNK):
            bank = k % 2
            if out_waits[bank] is not None:
                out_waits[bank].wait()

            def group(g, carry, k=k, bank=bank):
                for s in range(NBUF):
                    j = jnp.int32(k * STG) + g * NBUF + s
                    waits[s].wait()
                    process(j, s, bank)
                    fire(j + NBUF, s)
                return carry
            last = (k == NCHUNK - 1)
            ngrp = STG // NBUF - (1 if last else 0)
            lax.fori_loop(0, ngrp, group, 0)
            if last:
                for s in range(NBUF):
                    j = k * STG + (STG // NBUF - 1) * NBUF + s
                    waits[s].wait()
                    process(j, s, bank)
            out_waits[bank] = pltpu.async_copy(
                staging.at[bank], out.at[:, pl.ds(base + k * STG, STG)],
                out_sems[bank])
        for w in out_waits:
            w.wait()

    run_table(user_table_t, user_ids, out_u)
    run_table(item_table_t, item_ids, out_i)


def kernel(user_ids, item_ids, user_table, item_table):
    out_u_t, out_i_t = _embed_norm(user_ids.astype(jnp.int32),
                                   item_ids.astype(jnp.int32),
                                   user_table.T, item_table.T)
    return out_u_t.T, out_i_t.T
